# Initial kernel scaffold; baseline (speedup 1.0000x reference)
import jax
import jax.numpy as jnp
from jax.experimental import pallas as pl


def kernel(pos, bias):
    raise NotImplementedError("write your pallas kernel here")


# speedup vs baseline: 19.3147x; 19.3147x over previous
import functools

import numpy as np
import jax
import jax.numpy as jnp
from jax import lax
from jax.experimental import pallas as pl
from jax.experimental.pallas import tpu as pltpu
from jax.experimental.pallas import tpu_sc as plsc

NUM_HEADS = 16
NUM_BINS = 32
T = 2048
L = 16
NW = 32
ROWS_PER_W = T // NW
NJV = T // L

_MAX_DIST = np.float32(1000000.0)


def _bin_thresholds():
    dmax = np.float32(np.log1p(_MAX_DIST))

    def embin(d):
        r = np.log1p(np.float32(d), dtype=np.float32)
        s = np.float32(np.float32(r / dmax) * np.float32(NUM_BINS - 1))
        return int(np.int32(s))

    def f2i(x):
        return int(np.frombuffer(np.float32(x).tobytes(), dtype=np.int32)[0])

    def i2f(i):
        return np.frombuffer(np.int32(i).tobytes(), dtype=np.float32)[0]

    ths = []
    for b in range(1, NUM_BINS):
        lo, hi = 0, f2i(_MAX_DIST)
        while hi - lo > 1:
            mid = (lo + hi) // 2
            if embin(i2f(mid)) >= b:
                hi = mid
            else:
                lo = mid
        ths.append(float(i2f(hi)))
    return ths


_THRESH = _bin_thresholds()


def _make_sc_kernel():
    mesh = plsc.VectorSubcoreMesh(core_axis_name="c", subcore_axis_name="s")

    @functools.partial(
        pl.kernel,
        mesh=mesh,
        out_type=jax.ShapeDtypeStruct((NUM_HEADS, T, T), jnp.float32),
        scratch_types=[
            pltpu.VMEM((T,), jnp.float32),
            pltpu.VMEM((NUM_HEADS * NUM_BINS,), jnp.float32),
            pltpu.VMEM((NUM_HEADS, 1, T), jnp.float32),
        ],
        compiler_params=pltpu.CompilerParams(needs_layout_passes=False),
    )
    def k(pos_hbm, tab_hbm, out_hbm, pos_v, tab_v, buf_v):
        c = lax.axis_index("c")
        s = lax.axis_index("s")
        wid = s * 2 + c
        pltpu.sync_copy(pos_hbm, pos_v)
        pltpu.sync_copy(tab_hbm, tab_v)
        base = wid * ROWS_PER_W

        def row_body(r, carry):
            i = base + r
            pi = plsc.load_gather(pos_v, [jnp.full((L,), i, jnp.int32)])

            def jv_body(jv, carry2):
                pj = pos_v[pl.ds(jv * L, L)]
                d = jnp.abs(pi - pj)
                b = jnp.zeros((L,), jnp.int32)
                for t in _THRESH:
                    b = b + jnp.where(d >= t, 1, 0)
                for h in range(NUM_HEADS):
                    vals = plsc.load_gather(tab_v, [b + (h * NUM_BINS)])
                    buf_v[h, 0, pl.ds(jv * L, L)] = vals
                return carry2

            lax.fori_loop(0, NJV, jv_body, 0)
            pltpu.sync_copy(buf_v, out_hbm.at[:, pl.ds(i, 1), :])
            return carry

        lax.fori_loop(0, ROWS_PER_W, row_body, 0)

    return k


_sc_kernel = _make_sc_kernel()


def kernel(pos, bias):
    posf = pos.reshape(T)
    tab = bias.reshape(NUM_HEADS * NUM_BINS)
    out = _sc_kernel(posf, tab)
    return out[None]


# speedup vs baseline: 26.0710x; 1.3498x over previous
import functools

import numpy as np
import jax
import jax.numpy as jnp
from jax import lax
from jax.experimental import pallas as pl
from jax.experimental.pallas import tpu as pltpu
from jax.experimental.pallas import tpu_sc as plsc

NUM_HEADS = 16
NUM_BINS = 32
T = 2048
L = 16
NW = 32
ROWS_PER_W = T // NW
NJV = T // L
ETAB = 160

_MAX_DIST = np.float32(1000000.0)


def _f2i(x):
    return int(np.frombuffer(np.float32(x).tobytes(), dtype=np.int32)[0])


def _i2f(i):
    return np.frombuffer(np.int32(i).tobytes(), dtype=np.float32)[0]


def _build_tables():
    dmax = np.float32(np.log1p(_MAX_DIST))

    def embin(d):
        r = np.log1p(np.float32(d), dtype=np.float32)
        s = np.float32(np.float32(r / dmax) * np.float32(NUM_BINS - 1))
        return int(np.int32(s))

    ths = []
    for b in range(1, NUM_BINS):
        lo, hi = 0, _f2i(_MAX_DIST)
        while hi - lo > 1:
            mid = (lo + hi) // 2
            if embin(_i2f(mid)) >= b:
                hi = mid
            else:
                lo = mid
        ths.append(_i2f(hi))
    ths = np.array(ths, np.float32)

    def expo(d):
        return _f2i(np.float32(np.float32(1.0) + np.float32(d))) >> 23

    ebmax = expo(_i2f(_f2i(_MAX_DIST)))
    dmin = {}
    for eb in range(127, ebmax + 1):
        lo, hi = 0, _f2i(_MAX_DIST)
        if expo(_i2f(lo)) >= eb:
            dmin[eb] = 0.0
            continue
        while hi - lo > 1:
            mid = (lo + hi) // 2
            if expo(_i2f(mid)) >= eb:
                hi = mid
            else:
                lo = mid
        dmin[eb] = _i2f(hi)

    big = np.float32(3.0e38)
    blo = np.zeros(ETAB, np.int32)
    ta = np.full(ETAB, big, np.float32)
    tb = np.full(ETAB, big, np.float32)
    for eb in range(127, ebmax + 1):
        dlo = np.float32(dmin[eb])
        dhi = np.float32(dmin[eb + 1]) if eb + 1 in dmin else np.float32(2) * _MAX_DIST
        blo[eb] = int((dlo >= ths).sum())
        inside = ths[(ths > dlo) & (ths < dhi)]
        assert len(inside) <= 2
        if len(inside) >= 1:
            ta[eb] = inside[0]
        if len(inside) >= 2:
            tb[eb] = inside[1]
    return blo, ta, tb


_BLO, _TA, _TB = _build_tables()


def _make_sc_kernel():
    mesh = plsc.VectorSubcoreMesh(core_axis_name="c", subcore_axis_name="s")

    @functools.partial(
        pl.kernel,
        mesh=mesh,
        out_type=jax.ShapeDtypeStruct((NUM_HEADS, T, T), jnp.float32),
        scratch_types=[
            pltpu.VMEM((T,), jnp.float32),
            pltpu.VMEM((NUM_HEADS * NUM_BINS,), jnp.float32),
            pltpu.VMEM((ETAB,), jnp.int32),
            pltpu.VMEM((ETAB,), jnp.float32),
            pltpu.VMEM((ETAB,), jnp.float32),
            pltpu.VMEM((NUM_HEADS, 1, T), jnp.float32),
            pltpu.VMEM((NUM_HEADS, 1, T), jnp.float32),
            pltpu.SemaphoreType.DMA,
            pltpu.SemaphoreType.DMA,
        ],
        compiler_params=pltpu.CompilerParams(needs_layout_passes=False),
    )
    def k(pos_hbm, tab_hbm, blo_hbm, ta_hbm, tb_hbm, out_hbm,
          pos_v, tab_v, blo_v, ta_v, tb_v, buf0, buf1, sem0, sem1):
        c = lax.axis_index("c")
        s = lax.axis_index("s")
        wid = s * 2 + c
        pltpu.sync_copy(pos_hbm, pos_v)
        pltpu.sync_copy(tab_hbm, tab_v)
        pltpu.sync_copy(blo_hbm, blo_v)
        pltpu.sync_copy(ta_hbm, ta_v)
        pltpu.sync_copy(tb_hbm, tb_v)
        base = wid * ROWS_PER_W
        bufs = (buf0, buf1)
        sems = (sem0, sem1)

        def fill_row(i, buf):
            pi = plsc.load_gather(pos_v, [jnp.full((L,), i, jnp.int32)])

            def jv_body(jv, carry):
                pj = pos_v[pl.ds(jv * L, L)]
                d = jnp.abs(pi - pj)
                eb = lax.shift_right_logical(
                    plsc.bitcast(d + jnp.float32(1.0), jnp.int32), 23)
                b0 = plsc.load_gather(blo_v, [eb])
                tav = plsc.load_gather(ta_v, [eb])
                tbv = plsc.load_gather(tb_v, [eb])
                b = b0 + jnp.where(d >= tav, 1, 0) + jnp.where(d >= tbv, 1, 0)
                for h in range(NUM_HEADS):
                    vals = plsc.load_gather(tab_v, [b + (h * NUM_BINS)])
                    buf[h, 0, pl.ds(jv * L, L)] = vals
                return carry

            lax.fori_loop(0, NJV, jv_body, 0)

        def pair_body(p, carry):
            for bsel in range(2):
                i = base + p * 2 + bsel

                @pl.when(p > 0)
                def _wait():
                    pltpu.make_async_copy(
                        bufs[bsel], out_hbm.at[:, pl.ds(i, 1), :], sems[bsel]
                    ).wait()

                fill_row(i, bufs[bsel])
                pltpu.async_copy(
                    bufs[bsel], out_hbm.at[:, pl.ds(i, 1), :], sems[bsel])
            return carry

        lax.fori_loop(0, ROWS_PER_W // 2, pair_body, 0)
        for bsel in range(2):
            i = base + ROWS_PER_W - 2 + bsel
            pltpu.make_async_copy(
                bufs[bsel], out_hbm.at[:, pl.ds(i, 1), :], sems[bsel]
            ).wait()

    return k


_sc_kernel = _make_sc_kernel()


def kernel(pos, bias):
    posf = pos.reshape(T)
    tab = bias.reshape(NUM_HEADS * NUM_BINS)
    out = _sc_kernel(posf, tab, jnp.asarray(_BLO), jnp.asarray(_TA),
                     jnp.asarray(_TB))
    return out[None]


# speedup vs baseline: 63.2347x; 2.4255x over previous
import functools

import numpy as np
import jax
import jax.numpy as jnp
from jax import lax
from jax.experimental import pallas as pl
from jax.experimental.pallas import tpu as pltpu
from jax.experimental.pallas import tpu_sc as plsc

NUM_HEADS = 16
NUM_BINS = 32
T = 2048
L = 16
NW = 32
ROWS_PER_W = T // NW
NJV = T // L
ETAB = 160

_MAX_DIST = np.float32(1000000.0)


def _f2i(x):
    return int(np.frombuffer(np.float32(x).tobytes(), dtype=np.int32)[0])


def _i2f(i):
    return np.frombuffer(np.int32(i).tobytes(), dtype=np.float32)[0]


def _build_tables():
    dmax = np.float32(np.log1p(_MAX_DIST))

    def embin(d):
        r = np.log1p(np.float32(d), dtype=np.float32)
        s = np.float32(np.float32(r / dmax) * np.float32(NUM_BINS - 1))
        return int(np.int32(s))

    ths = []
    for b in range(1, NUM_BINS):
        lo, hi = 0, _f2i(_MAX_DIST)
        while hi - lo > 1:
            mid = (lo + hi) // 2
            if embin(_i2f(mid)) >= b:
                hi = mid
            else:
                lo = mid
        ths.append(_i2f(hi))
    ths = np.array(ths, np.float32)

    def expo(d):
        return _f2i(np.float32(np.float32(1.0) + np.float32(d))) >> 23

    ebmax = expo(_i2f(_f2i(_MAX_DIST)))
    dmin = {}
    for eb in range(127, ebmax + 1):
        lo, hi = 0, _f2i(_MAX_DIST)
        if expo(_i2f(lo)) >= eb:
            dmin[eb] = 0.0
            continue
        while hi - lo > 1:
            mid = (lo + hi) // 2
            if expo(_i2f(mid)) >= eb:
                hi = mid
            else:
                lo = mid
        dmin[eb] = _i2f(hi)

    big = np.float32(3.0e38)
    blo = np.zeros(ETAB, np.int32)
    ta = np.full(ETAB, big, np.float32)
    tb = np.full(ETAB, big, np.float32)
    for eb in range(127, ebmax + 1):
        dlo = np.float32(dmin[eb])
        dhi = np.float32(dmin[eb + 1]) if eb + 1 in dmin else np.float32(2) * _MAX_DIST
        blo[eb] = int((dlo >= ths).sum())
        inside = ths[(ths > dlo) & (ths < dhi)]
        assert len(inside) <= 2
        if len(inside) >= 1:
            ta[eb] = inside[0]
        if len(inside) >= 2:
            tb[eb] = inside[1]
    return blo, ta, tb


_BLO, _TA, _TB = _build_tables()


def _make_sc_kernel():
    mesh = plsc.VectorSubcoreMesh(core_axis_name="c", subcore_axis_name="s")

    @functools.partial(
        pl.kernel,
        mesh=mesh,
        out_type=jax.ShapeDtypeStruct((NUM_HEADS, T, T), jnp.float32),
        scratch_types=[
            pltpu.VMEM((T,), jnp.float32),
            pltpu.VMEM((NUM_HEADS * NUM_BINS,), jnp.float32),
            pltpu.VMEM((ETAB,), jnp.int32),
            pltpu.VMEM((ETAB,), jnp.float32),
            pltpu.VMEM((ETAB,), jnp.float32),
            pltpu.VMEM((NUM_HEADS, 1, T), jnp.float32),
            pltpu.VMEM((NUM_HEADS, 1, T), jnp.float32),
            pltpu.SemaphoreType.DMA,
            pltpu.SemaphoreType.DMA,
        ],
        compiler_params=pltpu.CompilerParams(needs_layout_passes=False),
    )
    def k(pos_hbm, tab_hbm, blo_hbm, ta_hbm, tb_hbm, out_hbm,
          pos_v, tab_v, blo_v, ta_v, tb_v, buf0, buf1, sem0, sem1):
        c = lax.axis_index("c")
        s = lax.axis_index("s")
        wid = s * 2 + c
        pltpu.sync_copy(pos_hbm, pos_v)
        pltpu.sync_copy(tab_hbm, tab_v)
        pltpu.sync_copy(blo_hbm, blo_v)
        pltpu.sync_copy(ta_hbm, ta_v)
        pltpu.sync_copy(tb_hbm, tb_v)
        base = wid * ROWS_PER_W
        bufs = (buf0, buf1)
        sems = (sem0, sem1)

        def fill_row(i, buf):
            pi = plsc.load_gather(pos_v, [jnp.full((L,), i, jnp.int32)])

            @plsc.parallel_loop(0, NJV, unroll=4)
            def jv_body(jv):
                pj = pos_v[pl.ds(jv * L, L)]
                d = jnp.abs(pi - pj)
                eb = lax.shift_right_logical(
                    plsc.bitcast(d + jnp.float32(1.0), jnp.int32), 23)
                b0 = plsc.load_gather(blo_v, [eb])
                tav = plsc.load_gather(ta_v, [eb])
                tbv = plsc.load_gather(tb_v, [eb])
                b = b0 + jnp.where(d >= tav, 1, 0) + jnp.where(d >= tbv, 1, 0)
                for h in range(NUM_HEADS):
                    vals = plsc.load_gather(tab_v, [b + (h * NUM_BINS)])
                    buf[h, 0, pl.ds(jv * L, L)] = vals

        def pair_body(p, carry):
            for bsel in range(2):
                i = base + p * 2 + bsel

                @pl.when(p > 0)
                def _wait():
                    pltpu.make_async_copy(
                        bufs[bsel], out_hbm.at[:, pl.ds(i, 1), :], sems[bsel]
                    ).wait()

                fill_row(i, bufs[bsel])
                pltpu.async_copy(
                    bufs[bsel], out_hbm.at[:, pl.ds(i, 1), :], sems[bsel])
            return carry

        lax.fori_loop(0, ROWS_PER_W // 2, pair_body, 0)
        for bsel in range(2):
            i = base + ROWS_PER_W - 2 + bsel
            pltpu.make_async_copy(
                bufs[bsel], out_hbm.at[:, pl.ds(i, 1), :], sems[bsel]
            ).wait()

    return k


_sc_kernel = _make_sc_kernel()


def kernel(pos, bias):
    posf = pos.reshape(T)
    tab = bias.reshape(NUM_HEADS * NUM_BINS)
    out = _sc_kernel(posf, tab, jnp.asarray(_BLO), jnp.asarray(_TA),
                     jnp.asarray(_TB))
    return out[None]
